# baseline (speedup 1.0000x reference)
import jax
import jax.numpy as jnp
from jax import lax
from jax.experimental import pallas as pl

TOKENS = 32768
D_MODEL = 768
NUM_EXPERTS = 64
BLOCK = 4096


def _gate_body(x_ref, w_ref, bt_ref, o_ref):
    scores_t = lax.dot_general(
        w_ref[...],
        x_ref[...],
        (((1,), (1,)), ((), ())),
        preferred_element_type=jnp.float32,
    )
    scores_t = scores_t + bt_ref[...]
    m = jnp.max(scores_t, axis=0, keepdims=True)
    row = lax.broadcasted_iota(jnp.int32, scores_t.shape, 0)
    idx = jnp.min(jnp.where(scores_t == m, row, NUM_EXPERTS), axis=0, keepdims=True)
    oh_t = (row == idx).astype(jnp.float32)
    e1 = lax.broadcasted_iota(jnp.int32, (NUM_EXPERTS, NUM_EXPERTS), 0)
    e2 = lax.broadcasted_iota(jnp.int32, (NUM_EXPERTS, NUM_EXPERTS), 1)
    eye = (e1 == e2).astype(jnp.float32)
    o_ref[...] = lax.dot_general(
        oh_t, eye, (((0,), (0,)), ((), ())), preferred_element_type=jnp.float32
    )


def kernel(x, W, b):
    bt = b.reshape(NUM_EXPERTS, 1)
    return pl.pallas_call(
        _gate_body,
        grid=(TOKENS // BLOCK,),
        in_specs=[
            pl.BlockSpec((BLOCK, D_MODEL), lambda i: (i, 0)),
            pl.BlockSpec((NUM_EXPERTS, D_MODEL), lambda i: (0, 0)),
            pl.BlockSpec((NUM_EXPERTS, 1), lambda i: (0, 0)),
        ],
        out_specs=pl.BlockSpec((BLOCK, NUM_EXPERTS), lambda i: (i, 0)),
        out_shape=jax.ShapeDtypeStruct((TOKENS, NUM_EXPERTS), jnp.float32),
    )(x, W, bt)
